# Initial kernel scaffold; baseline (speedup 1.0000x reference)
import jax
import jax.numpy as jnp
from jax.experimental import pallas as pl


def kernel(inputs, k_inputs, targets, features):
    raise NotImplementedError("write your pallas kernel here")


# speedup vs baseline: 1.6339x; 1.6339x over previous
import jax
import jax.numpy as jnp
from jax.experimental import pallas as pl

_B = 4096
_D = 128
_S = 16384
_TEMP = 0.05
_INV_TEMP = 1.0 / _TEMP
_SHIFT = _INV_TEMP

_BB = 512
_SC = 2048
_NB = _B // _BB
_NS = _S // _SC


def _ce_kernel(x_ref, t_ref, f_ref, out_ref):
    i = pl.program_id(0)
    x = x_ref[...]
    nrm = jnp.sqrt(jnp.sum(x * x, axis=1, keepdims=True))
    xn = x / jnp.maximum(nrm, 1e-12)
    tgt_idx = t_ref[0]

    def body(j, carry):
        acc, tlogit = carry
        f = f_ref[pl.ds(j * _SC, _SC), :]
        l = jax.lax.dot_general(
            xn, f, (((1,), (1,)), ((), ())),
            preferred_element_type=jnp.float32) * _INV_TEMP
        acc = acc + jnp.sum(jnp.exp(l - _SHIFT), axis=1, keepdims=True)
        col = jax.lax.broadcasted_iota(jnp.int32, (_BB, _SC), 1) + j * _SC
        hit = col == tgt_idx
        tlogit = tlogit + jnp.sum(jnp.where(hit, l, 0.0), axis=1, keepdims=True)
        return acc, tlogit

    acc0 = jnp.zeros((_BB, 1), jnp.float32)
    acc, tlogit = jax.lax.fori_loop(0, _NS, body, (acc0, acc0))
    partial = jnp.sum(_SHIFT + jnp.log(acc) - tlogit).reshape(1, 1)

    @pl.when(i == 0)
    def _():
        out_ref[...] = partial

    @pl.when(i > 0)
    def _():
        out_ref[...] = out_ref[...] + partial


def kernel(inputs, k_inputs, targets, features):
    del k_inputs
    t3 = targets.astype(jnp.int32).reshape(_NB, _BB, 1)
    out = pl.pallas_call(
        _ce_kernel,
        grid=(_NB,),
        in_specs=[
            pl.BlockSpec((_BB, _D), lambda i: (i, 0)),
            pl.BlockSpec((1, _BB, 1), lambda i: (i, 0, 0)),
            pl.BlockSpec((_S, _D), lambda i: (0, 0)),
        ],
        out_specs=pl.BlockSpec((1, 1), lambda i: (0, 0)),
        out_shape=jax.ShapeDtypeStruct((1, 1), jnp.float32),
    )(inputs, t3, features)
    return out[0, 0] / _B
